# baseline (speedup 1.0000x reference)
import functools

import numpy as np
import jax
import jax.numpy as jnp
from jax import lax
from jax.experimental import pallas as pl
from jax.experimental.pallas import tpu as pltpu
from jax.experimental.pallas import tpu_sc as plsc

_B = 4096
_F = 26
_V = 100000
_D = 64
_NF = _F + 1

_NC, _NS = 2, 16
_NW = _NC * _NS
_ROWS = _F * _B
_CHUNK = 128
_NCHUNK = _ROWS // (_NW * _CHUNK)
_FPERROW = _B // _CHUNK
_VP = _V // 2


def _sc_gather(pair_tab, lsi3d):
    mesh = plsc.VectorSubcoreMesh(core_axis_name="c", subcore_axis_name="s")

    @functools.partial(
        pl.kernel,
        mesh=mesh,
        out_type=jax.ShapeDtypeStruct((_ROWS, 2 * _D), jnp.float32),
        scratch_types=[
            pltpu.VMEM((_NCHUNK, _CHUNK), jnp.int32),
            pltpu.VMEM((_CHUNK, 2 * _D), jnp.float32),
            pltpu.VMEM((_CHUNK, 2 * _D), jnp.float32),
            pltpu.SemaphoreType.DMA,
            pltpu.SemaphoreType.DMA,
        ],
    )
    def k(tab_hbm, lsi_hbm, out_hbm, idx_v, rows_a, rows_b, sem_a, sem_b):
        wid = lax.axis_index("s") * _NC + lax.axis_index("c")
        row0 = wid * _NCHUNK
        pltpu.sync_copy(lsi_hbm.at[wid], idx_v)
        for c in range(_NCHUNK):
            off = ((row0 + c) // _FPERROW) * _VP
            for g in range(_CHUNK // 16):
                sl = (c, pl.ds(g * 16, 16))
                idx_v[sl] = (idx_v[sl] >> 1) + off
        bufs = ((rows_a, sem_a), (rows_b, sem_b))
        cps = []
        for c in range(_NCHUNK):
            buf, sem = bufs[c % 2]
            cps.append(pltpu.async_copy(tab_hbm.at[idx_v.at[c]], buf, sem))
            if c >= 1:
                pbuf, _ = bufs[(c - 1) % 2]
                cps[c - 1].wait()
                pltpu.sync_copy(
                    pbuf, out_hbm.at[pl.ds((row0 + c - 1) * _CHUNK, _CHUNK)]
                )
        cps[-1].wait()
        pltpu.sync_copy(
            bufs[(_NCHUNK - 1) % 2][0],
            out_hbm.at[pl.ds((row0 + _NCHUNK - 1) * _CHUNK, _CHUNK)],
        )

    return k(pair_tab, lsi3d)


_BB = 512
_NBLK = _B // _BB


def _tc_body(xref, lyref, lsiref, wb0, bb0, wb1, bb1, wb2, bb2,
             wt0a, wt0p, bt0, wt1, bt1, wt2, bt2, oref):
    f32 = jnp.float32
    x = xref[...]
    x = jnp.maximum(jnp.dot(x, wb0[...], preferred_element_type=f32) + bb0[...], 0.0)
    x = jnp.maximum(jnp.dot(x, wb1[...], preferred_element_type=f32) + bb1[...], 0.0)
    x = jnp.maximum(jnp.dot(x, wb2[...], preferred_element_type=f32) + bb2[...], 0.0)
    ly128 = lyref[...]
    par = (lsiref[...] & 1)[:, :, None] == 1
    ly = jnp.where(par, ly128[:, :, _D:], ly128[:, :, :_D])
    tt = jnp.concatenate([x[None], ly], axis=0)
    cols = [jnp.sum(tt * tt[i][None], axis=-1) for i in range(_NF)]
    zffm = jnp.concatenate(cols, axis=0)
    h = jnp.dot(x, wt0a[...], preferred_element_type=f32)
    h = h + lax.dot_general(zffm, wt0p[...], (((0,), (0,)), ((), ())),
                            preferred_element_type=f32)
    h = jnp.maximum(h + bt0[...], 0.0)
    h = jnp.maximum(jnp.dot(h, wt1[...], preferred_element_type=f32) + bt1[...], 0.0)
    z = jnp.dot(h, wt2[...], preferred_element_type=f32) + bt2[...]
    oref[...] = 1.0 / (1.0 + jnp.exp(-z))


def _full(shape):
    nd = len(shape)
    return pl.BlockSpec(shape, lambda i, _nd=nd: (0,) * _nd)


def kernel(dense_x, lS_i, W_emb, Wb0, bb0, Wb1, bb1, Wb2, bb2,
           Wt0, bt0, Wt1, bt1, Wt2, bt2):
    pair_tab = jnp.concatenate(
        [W_emb[:, 0::2, :], W_emb[:, 1::2, :]], axis=2
    ).reshape(_F * _VP, 2 * _D)
    lsi3d = lS_i.reshape(_NW, _NCHUNK, _CHUNK)
    gathered = _sc_gather(pair_tab, lsi3d)
    ly128 = gathered.reshape(_F, _B, 2 * _D)
    pairs = np.array([i * _NF + j for i in range(_NF) for j in range(i)],
                     dtype=np.int32)
    wt0p = jnp.zeros((_NF * _NF, 512), jnp.float32).at[pairs].set(Wt0[:, 64:].T)
    out = pl.pallas_call(
        _tc_body,
        grid=(_NBLK,),
        in_specs=[
            pl.BlockSpec((_BB, 13), lambda i: (i, 0)),
            pl.BlockSpec((_F, _BB, 2 * _D), lambda i: (0, i, 0)),
            pl.BlockSpec((_F, _BB), lambda i: (0, i)),
            _full((13, 512)), _full((512,)),
            _full((512, 256)), _full((256,)),
            _full((256, 64)), _full((64,)),
            _full((64, 512)), _full((_NF * _NF, 512)), _full((512,)),
            _full((512, 256)), _full((256,)),
            _full((256, 1)), _full((1,)),
        ],
        out_specs=pl.BlockSpec((_BB, 1), lambda i: (i, 0)),
        out_shape=jax.ShapeDtypeStruct((_B, 1), jnp.float32),
    )(dense_x, ly128, lS_i, Wb0.T, bb0, Wb1.T, bb1, Wb2.T, bb2,
      Wt0[:, :64].T, wt0p, bt0, Wt1.T, bt1, Wt2.T, bt2)
    return out


# speedup vs baseline: 14.2511x; 14.2511x over previous
import functools

import numpy as np
import jax
import jax.numpy as jnp
from jax import lax
from jax.experimental import pallas as pl
from jax.experimental.pallas import tpu as pltpu
from jax.experimental.pallas import tpu_sc as plsc

_B = 4096
_F = 26
_V = 100000
_D = 64
_NF = _F + 1

_NC, _NS = 2, 16
_NW = _NC * _NS
_ROWS = _F * _B
_CHUNK = 128
_NCHUNK = _ROWS // (_NW * _CHUNK)
_FPERROW = _B // _CHUNK
_VP = _V // 2


def _sc_gather(pair_tab, lsi3d):
    mesh = plsc.VectorSubcoreMesh(core_axis_name="c", subcore_axis_name="s")

    @functools.partial(
        pl.kernel,
        mesh=mesh,
        out_type=jax.ShapeDtypeStruct((_ROWS, 2 * _D), jnp.float32),
        scratch_types=[
            pltpu.VMEM((_NCHUNK, _CHUNK), jnp.int32),
            pltpu.VMEM((_CHUNK, 2 * _D), jnp.float32),
            pltpu.VMEM((_CHUNK, 2 * _D), jnp.float32),
            pltpu.SemaphoreType.DMA,
            pltpu.SemaphoreType.DMA,
        ],
    )
    def k(tab_hbm, lsi_hbm, out_hbm, idx_v, rows_a, rows_b, sem_a, sem_b):
        wid = lax.axis_index("s") * _NC + lax.axis_index("c")
        row0 = wid * _NCHUNK
        pltpu.sync_copy(lsi_hbm.at[wid], idx_v)
        for c in range(_NCHUNK):
            off = ((row0 + c) // _FPERROW) * _VP
            for g in range(_CHUNK // 16):
                sl = (c, pl.ds(g * 16, 16))
                v = idx_v[sl]
                idx_v[sl] = v - jnp.where(v >= _VP, _VP, 0) + off
        bufs = ((rows_a, sem_a), (rows_b, sem_b))
        cps = []
        for c in range(_NCHUNK):
            buf, sem = bufs[c % 2]
            cps.append(pltpu.async_copy(tab_hbm.at[idx_v.at[c]], buf, sem))
            if c >= 1:
                pbuf, _ = bufs[(c - 1) % 2]
                cps[c - 1].wait()
                pltpu.sync_copy(
                    pbuf, out_hbm.at[pl.ds((row0 + c - 1) * _CHUNK, _CHUNK)]
                )
        cps[-1].wait()
        pltpu.sync_copy(
            bufs[(_NCHUNK - 1) % 2][0],
            out_hbm.at[pl.ds((row0 + _NCHUNK - 1) * _CHUNK, _CHUNK)],
        )

    return k(pair_tab, lsi3d)


_BB = 512
_NBLK = _B // _BB


def _tc_body(xref, lyref, lsiref, wb0, bb0, wb1, bb1, wb2, bb2,
             wt0a, wt0p, bt0, wt1, bt1, wt2, bt2, oref):
    f32 = jnp.float32
    x = xref[...]
    x = jnp.maximum(jnp.dot(x, wb0[...], preferred_element_type=f32) + bb0[...], 0.0)
    x = jnp.maximum(jnp.dot(x, wb1[...], preferred_element_type=f32) + bb1[...], 0.0)
    x = jnp.maximum(jnp.dot(x, wb2[...], preferred_element_type=f32) + bb2[...], 0.0)
    ly128 = lyref[...]
    par = lsiref[...][:, :, None] >= _VP
    ly = jnp.where(par, ly128[:, :, _D:], ly128[:, :, :_D])
    tt = jnp.concatenate([x[None], ly], axis=0)
    cols = [jnp.sum(tt * tt[i][None], axis=-1) for i in range(_NF)]
    zffm = jnp.concatenate(cols, axis=0)
    h = jnp.dot(x, wt0a[...], preferred_element_type=f32)
    h = h + lax.dot_general(zffm, wt0p[...], (((0,), (0,)), ((), ())),
                            preferred_element_type=f32)
    h = jnp.maximum(h + bt0[...], 0.0)
    h = jnp.maximum(jnp.dot(h, wt1[...], preferred_element_type=f32) + bt1[...], 0.0)
    z = jnp.dot(h, wt2[...], preferred_element_type=f32) + bt2[...]
    oref[...] = 1.0 / (1.0 + jnp.exp(-z))


def _full(shape):
    nd = len(shape)
    return pl.BlockSpec(shape, lambda i, _nd=nd: (0,) * _nd)


def kernel(dense_x, lS_i, W_emb, Wb0, bb0, Wb1, bb1, Wb2, bb2,
           Wt0, bt0, Wt1, bt1, Wt2, bt2):
    pair_tab = jnp.concatenate(
        [W_emb[:, :_VP, :], W_emb[:, _VP:, :]], axis=2
    ).reshape(_F * _VP, 2 * _D)
    lsi3d = lS_i.reshape(_NW, _NCHUNK, _CHUNK)
    gathered = _sc_gather(pair_tab, lsi3d)
    ly128 = gathered.reshape(_F, _B, 2 * _D)
    pairs = np.array([i * _NF + j for i in range(_NF) for j in range(i)],
                     dtype=np.int32)
    wt0p = jnp.zeros((_NF * _NF, 512), jnp.float32).at[pairs].set(Wt0[:, 64:].T)
    out = pl.pallas_call(
        _tc_body,
        grid=(_NBLK,),
        in_specs=[
            pl.BlockSpec((_BB, 13), lambda i: (i, 0)),
            pl.BlockSpec((_F, _BB, 2 * _D), lambda i: (0, i, 0)),
            pl.BlockSpec((_F, _BB), lambda i: (0, i)),
            _full((13, 512)), _full((512,)),
            _full((512, 256)), _full((256,)),
            _full((256, 64)), _full((64,)),
            _full((64, 512)), _full((_NF * _NF, 512)), _full((512,)),
            _full((512, 256)), _full((256,)),
            _full((256, 1)), _full((1,)),
        ],
        out_specs=pl.BlockSpec((_BB, 1), lambda i: (i, 0)),
        out_shape=jax.ShapeDtypeStruct((_B, 1), jnp.float32),
    )(dense_x, ly128, lS_i, Wb0.T, bb0, Wb1.T, bb1, Wb2.T, bb2,
      Wt0[:, :64].T, wt0p, bt0, Wt1.T, bt1, Wt2.T, bt2)
    return out
